# Initial kernel scaffold; baseline (speedup 1.0000x reference)
import jax
import jax.numpy as jnp
from jax.experimental import pallas as pl


def kernel(queries, keys):
    raise NotImplementedError("write your pallas kernel here")


# speedup vs baseline: 2.9409x; 2.9409x over previous
import jax
import jax.numpy as jnp
from jax.experimental import pallas as pl
from jax.experimental.pallas import tpu as pltpu

_Q = 1024
_D = 768
_K = 100000
_BK = 5000


def _topk_kernel(q_ref, k_ref, val_ref, idx_ref):
    j = pl.program_id(0)
    s = jax.lax.dot_general(
        q_ref[...], k_ref[...],
        dimension_numbers=(((1,), (1,)), ((), ())),
        preferred_element_type=jnp.float32,
    )
    bmax = jnp.max(s, axis=1, keepdims=True)
    barg = jnp.argmax(s, axis=1, keepdims=True)
    bidx = (barg + j * _BK).astype(jnp.int32)

    @pl.when(j == 0)
    def _init():
        val_ref[...] = bmax
        idx_ref[...] = bidx

    @pl.when(j > 0)
    def _update():
        prev = val_ref[...]
        take_new = bmax > prev
        val_ref[...] = jnp.where(take_new, bmax, prev)
        idx_ref[...] = jnp.where(take_new, bidx, idx_ref[...])


def kernel(queries, keys):
    grid = (_K // _BK,)
    top_vals, top_idx = pl.pallas_call(
        _topk_kernel,
        grid=grid,
        in_specs=[
            pl.BlockSpec((_Q, _D), lambda j: (0, 0)),
            pl.BlockSpec((_BK, _D), lambda j: (j, 0)),
        ],
        out_specs=[
            pl.BlockSpec((_Q, 1), lambda j: (0, 0)),
            pl.BlockSpec((_Q, 1), lambda j: (0, 0)),
        ],
        out_shape=[
            jax.ShapeDtypeStruct((_Q, 1), jnp.float32),
            jax.ShapeDtypeStruct((_Q, 1), jnp.int32),
        ],
        compiler_params=pltpu.CompilerParams(
            dimension_semantics=("arbitrary",),
        ),
    )(queries, keys)
    return top_vals, top_idx
